# baseline (speedup 1.0000x reference)
import functools

import jax
import jax.numpy as jnp
from jax import lax
from jax.experimental import pallas as pl
from jax.experimental.pallas import tpu as pltpu
from jax.experimental.pallas import tpu_sc as plsc

B = 4096
CON = 13
CAT = 26
V = 100000
D = 32
ROWS = 1 + CON + CAT

_INFO = plsc.get_sparse_core_info()
NC, NS, L = _INFO.num_cores, _INFO.num_subcores, _INFO.num_lanes
NW = NC * NS
BPW = B // NW
CH = 64
NCHUNK = BPW // CH
GID = CH * CAT
NGD = GID // 128
DEN = CH * (1 + CON)
NDD = DEN // 128


def _iota16():
    return lax.iota(jnp.int32, L)


def _sc_kernel(tab_hbm, xcat_hbm, xcon_hbm, wcon_hbm, bcon_hbm, cls_hbm,
               out_hbm,
               xcat_v, xcon_v, w_v, bb_v, cls_v,
               gidx_v, ocat_v, oden_v, rows_v, dense_v,
               gsem, ssem):
    wid = lax.axis_index("s") * NC + lax.axis_index("c")
    base_b = wid * BPW

    pltpu.sync_copy(wcon_hbm, w_v)
    pltpu.sync_copy(bcon_hbm, bb_v)
    pltpu.sync_copy(cls_hbm, cls_v)
    cls0 = cls_v[0, pl.ds(0, L)]
    cls1 = cls_v[0, pl.ds(L, L)]

    for ci in range(NCHUNK):
        b0 = base_b + ci * CH
        pltpu.sync_copy(
            xcat_hbm.at[pl.ds(pl.multiple_of(b0 * CAT, GID), GID)], xcat_v)
        pltpu.sync_copy(
            xcon_hbm.at[pl.ds(pl.multiple_of(b0, CH), CH)], xcon_v)

        ob = b0 * ROWS
        for j in range(NGD):
            for l in range(128 // L):
                k = j * 128 + l * L
                kk = _iota16() + k
                fld = kk % CAT
                gidx_v[j, pl.ds(l * L, L)] = xcat_v[pl.ds(k, L)] + fld * V
                bloc = lax.div(kk, jnp.full((L,), CAT, jnp.int32))
                ocat_v[j, pl.ds(l * L, L)] = ob + bloc * ROWS + (1 + CON) + fld
        for j in range(NDD):
            for l in range(128 // L):
                kk = _iota16() + j * 128 + l * L
                bloc = lax.div(kk, jnp.full((L,), 1 + CON, jnp.int32))
                oden_v[j, pl.ds(l * L, L)] = (
                    ob + bloc * ROWS + kk % (1 + CON))

        gathers = [
            pltpu.async_copy(tab_hbm.at[gidx_v.at[j]],
                             rows_v.at[pl.ds(j * 128, 128)], gsem)
            for j in range(NGD)
        ]

        def dense_body(bl, carry):
            row = bl * (1 + CON)
            dense_v[row, pl.ds(0, L)] = cls0
            dense_v[row, pl.ds(L, L)] = cls1
            xv = xcon_v[bl, :]
            for j in range(CON):
                x = lax.gather(
                    xv, jnp.full((L, 1), j, jnp.int32),
                    lax.GatherDimensionNumbers(
                        offset_dims=(), collapsed_slice_dims=(0,),
                        start_index_map=(0,)),
                    (1,), mode=lax.GatherScatterMode.PROMISE_IN_BOUNDS)
                r = row + 1 + j
                dense_v[r, pl.ds(0, L)] = jnp.maximum(
                    x * w_v[j, pl.ds(0, L)] + bb_v[j, pl.ds(0, L)], 0.0)
                dense_v[r, pl.ds(L, L)] = jnp.maximum(
                    x * w_v[j, pl.ds(L, L)] + bb_v[j, pl.ds(L, L)], 0.0)
            return carry

        lax.fori_loop(0, CH, dense_body, 0)

        for g in gathers:
            g.wait()
        scatters = [
            pltpu.async_copy(rows_v.at[pl.ds(j * 128, 128)],
                             out_hbm.at[ocat_v.at[j]], ssem)
            for j in range(NGD)
        ] + [
            pltpu.async_copy(dense_v.at[pl.ds(j * 128, 128)],
                             out_hbm.at[oden_v.at[j]], ssem)
            for j in range(NDD)
        ]
        for s in scatters:
            s.wait()


@functools.partial(
    pl.kernel,
    out_type=jax.ShapeDtypeStruct((B * ROWS, D), jnp.float32),
    mesh=plsc.VectorSubcoreMesh(core_axis_name="c", subcore_axis_name="s"),
    compiler_params=pltpu.CompilerParams(use_tc_tiling_on_sc=False),
    scratch_types=[
        pltpu.VMEM((GID,), jnp.int32),
        pltpu.VMEM((CH, L), jnp.float32),
        pltpu.VMEM((CON, D), jnp.float32),
        pltpu.VMEM((CON, D), jnp.float32),
        pltpu.VMEM((1, D), jnp.float32),
        pltpu.VMEM((NGD, 128), jnp.int32),
        pltpu.VMEM((NGD, 128), jnp.int32),
        pltpu.VMEM((NDD, 128), jnp.int32),
        pltpu.VMEM((GID, D), jnp.float32),
        pltpu.VMEM((DEN, D), jnp.float32),
        pltpu.SemaphoreType.DMA,
        pltpu.SemaphoreType.DMA,
    ],
)
def _graph_embed(*refs):
    _sc_kernel(*refs)


def kernel(x_con, x_cat, cls, W_con, b_con, tables):
    tab_flat = tables.reshape(CAT * V, D)
    xcat_flat = x_cat.reshape(B * CAT)
    xcon_pad = jnp.pad(x_con, ((0, 0), (0, L - CON)))
    out = _graph_embed(tab_flat, xcat_flat, xcon_pad, W_con, b_con, cls)
    return out.reshape(B, ROWS, D)


# speedup vs baseline: 1.8785x; 1.8785x over previous
import functools

import jax
import jax.numpy as jnp
from jax import lax
from jax.experimental import pallas as pl
from jax.experimental.pallas import tpu as pltpu
from jax.experimental.pallas import tpu_sc as plsc

B = 4096
CON = 13
CAT = 26
V = 100000
D = 32
ROWS = 1 + CON + CAT

_INFO = plsc.get_sparse_core_info()
NC, NS, L = _INFO.num_cores, _INFO.num_subcores, _INFO.num_lanes
NW = NC * NS
BPW = B // NW
NG = BPW // L


def _splat(vec, lane):
    return jnp.zeros((L,), vec.dtype) + vec[lane]


def _sc_kernel(tab_hbm, xcat_hbm, xcon_hbm, wcon_hbm, bcon_hbm, cls_hbm,
               out_hbm,
               idx_v, gbuf_v, dbuf_v, xc_v, w_v, bb_v, cls_v, gsem):
    wid = lax.axis_index("s") * NC + lax.axis_index("c")
    b0 = pl.multiple_of(wid * BPW, BPW)

    def field_body(f, carry):
        pltpu.sync_copy(xcat_hbm.at[f, pl.ds(b0, BPW)], idx_v)
        gathers = [
            pltpu.async_copy(tab_hbm.at[f * D + d].at[idx_v],
                             gbuf_v.at[d], gsem)
            for d in range(D)
        ]
        for g in gathers:
            g.wait()
        r0 = pl.multiple_of((1 + CON + f) * D, D)
        pltpu.sync_copy(gbuf_v, out_hbm.at[pl.ds(r0, D), pl.ds(b0, BPW)])
        return carry

    lax.fori_loop(0, CAT, field_body, 0)

    for j in range(CON):
        pltpu.sync_copy(xcon_hbm.at[j, pl.ds(b0, BPW)], xc_v.at[j])
    pltpu.sync_copy(wcon_hbm, w_v)
    pltpu.sync_copy(bcon_hbm, bb_v)
    pltpu.sync_copy(cls_hbm, cls_v)

    for d in range(D):
        sp = cls_v[pl.ds(d * L, L)]
        for g in range(NG):
            dbuf_v[d, pl.ds(g * L, L)] = sp
    pltpu.sync_copy(dbuf_v, out_hbm.at[pl.ds(0, D), pl.ds(b0, BPW)])

    def con_body(j, carry):
        xg = [xc_v[j, pl.ds(g * L, L)] for g in range(NG)]
        jo = pl.multiple_of(j * D * L, D * L)
        for d in range(D):
            w_s = w_v[pl.ds(jo + d * L, L)]
            b_s = bb_v[pl.ds(jo + d * L, L)]
            for g in range(NG):
                dbuf_v[d, pl.ds(g * L, L)] = jnp.maximum(
                    xg[g] * w_s + b_s, 0.0)
        r0 = pl.multiple_of((1 + j) * D, D)
        pltpu.sync_copy(dbuf_v, out_hbm.at[pl.ds(r0, D), pl.ds(b0, BPW)])
        return carry

    lax.fori_loop(0, CON, con_body, 0)


@functools.partial(
    pl.kernel,
    out_type=jax.ShapeDtypeStruct((ROWS * D, B), jnp.float32),
    mesh=plsc.VectorSubcoreMesh(core_axis_name="c", subcore_axis_name="s"),
    compiler_params=pltpu.CompilerParams(use_tc_tiling_on_sc=False),
    scratch_types=[
        pltpu.VMEM((BPW,), jnp.int32),
        pltpu.VMEM((D, BPW), jnp.float32),
        pltpu.VMEM((D, BPW), jnp.float32),
        pltpu.VMEM((CON, BPW), jnp.float32),
        pltpu.VMEM((CON * D * L,), jnp.float32),
        pltpu.VMEM((CON * D * L,), jnp.float32),
        pltpu.VMEM((D * L,), jnp.float32),
        pltpu.SemaphoreType.DMA,
    ],
)
def _graph_embed(*refs):
    _sc_kernel(*refs)


def kernel(x_con, x_cat, cls, W_con, b_con, tables):
    tabT = jnp.transpose(tables, (0, 2, 1)).reshape(CAT * D, V)
    wsp = jnp.broadcast_to(W_con[:, :, None], (CON, D, L)).reshape(-1)
    bsp = jnp.broadcast_to(b_con[:, :, None], (CON, D, L)).reshape(-1)
    csp = jnp.broadcast_to(cls[0, :, None], (D, L)).reshape(-1)
    out = _graph_embed(tabT, x_cat.T, x_con.T, wsp, bsp, csp)
    return out.reshape(ROWS, D, B).transpose(2, 0, 1)
